# baseline (speedup 1.0000x reference)
import functools

import jax
import jax.numpy as jnp
from jax import lax
from jax.experimental import pallas as pl
from jax.experimental.pallas import tpu as pltpu
from jax.experimental.pallas import tpu_sc as plsc

_NC, _NS = 2, 16
_NW = _NC * _NS


@functools.lru_cache(maxsize=None)
def _make_transpose(V, D):
    full_tc = V // 128
    rem = V - full_tc * 128
    base_per_w = full_tc // _NW
    extra = full_tc % _NW
    max_slabs = base_per_w + (1 if extra else 0)
    n_groups = (max_slabs + 1) // 2
    rows_out = V // 2 if rem == 0 else (V + 127) // 128 * 64
    mesh = plsc.VectorSubcoreMesh(
        core_axis_name="c", subcore_axis_name="s",
        num_cores=_NC, num_subcores=_NS,
    )

    @functools.partial(
        pl.kernel,
        out_type=jax.ShapeDtypeStruct((rows_out, 2 * D), jnp.float32),
        mesh=mesh,
        scratch_types=[
            pltpu.VMEM((D, 128), jnp.float32),
            pltpu.VMEM((D, 128), jnp.float32),
            pltpu.VMEM((64, 2 * D), jnp.float32),
            pltpu.VMEM((64, 2 * D), jnp.float32),
            pltpu.VMEM((rem // 2 if rem else 8, 128), jnp.float32),
            pltpu.SemaphoreType.DMA,
            pltpu.SemaphoreType.DMA,
            pltpu.SemaphoreType.DMA,
            pltpu.SemaphoreType.DMA,
        ],
        compiler_params=pltpu.CompilerParams(
            use_tc_tiling_on_sc=True, needs_layout_passes=False),
    )
    def k(wt_hbm, aux_hbm, out_hbm, buf0, buf1, st0, st1, bounce,
          si0, si1, so0, so1):
        wid = lax.axis_index("s") * _NC + lax.axis_index("c")
        n_w = base_per_w + jnp.where(wid < extra, 1, 0)
        start_w = base_per_w * wid + jnp.minimum(wid, extra)
        buf, st = [buf0, buf1], [st0, st1]
        si, so = [si0, si1], [so0, so1]

        row_idx = [lax.iota(jnp.int32, 16) + 16 * kk for kk in range(D // 16)]

        def in_copy(s, b):
            return pltpu.make_async_copy(
                wt_hbm.at[:, pl.ds((start_w + s) * 128, 128)], buf[b], si[b])

        def out_copy(s, b):
            return pltpu.make_async_copy(
                st[b], out_hbm.at[pl.ds((start_w + s) * 64, 64)], so[b])

        def transpose_slab(b):
            @pl.loop(0, 128)
            def _col(c):
                colv = jnp.full((16,), c, jnp.int32)
                r2 = c // 2
                o2 = (c % 2) * D
                for kk in range(D // 16):
                    v = plsc.load_gather(buf[b], [row_idx[kk], colv])
                    st[b][r2, pl.ds(o2 + 16 * kk, 16)] = v

        def do_slab(s, b, drain):
            in_copy(s, b).wait()

            @pl.when(s + 1 < n_w)
            def _nxt():
                in_copy(s + 1, 1 - b).start()

            if drain:
                out_copy(s - 2, b).wait()
            transpose_slab(b)
            out_copy(s, b).start()

        in_copy(0, 0).start()
        for b in range(2):
            @pl.when(b < n_w)
            def _first(b=b):
                do_slab(b, b, drain=False)

        @pl.loop(1, n_groups)
        def _group(g):
            for b in range(2):
                s = 2 * g + b

                @pl.when(s < n_w)
                def _do(s=s, b=b):
                    do_slab(s, b, drain=True)

        for b in range(2):
            out_copy(0, b).wait()

        if rem:
            @pl.when(wid == _NW - 1)
            def _aux():
                pltpu.sync_copy(aux_hbm, bounce)
                pltpu.sync_copy(bounce, out_hbm.at[pl.ds(full_tc * 64, rem // 2)])

    return k


@functools.lru_cache(maxsize=None)
def _make_gather(B, D, C):
    b_per_w = B // _NW
    n_chunks = b_per_w // C
    mesh = plsc.VectorSubcoreMesh(
        core_axis_name="c", subcore_axis_name="s",
        num_cores=_NC, num_subcores=_NS,
    )

    @functools.partial(
        pl.kernel,
        out_type=jax.ShapeDtypeStruct((B, D), jnp.float32),
        mesh=mesh,
        scratch_types=[
            pltpu.VMEM((b_per_w,), jnp.int32),
            pltpu.VMEM((C, D), jnp.float32),
            pltpu.VMEM((C, D), jnp.float32),
            pltpu.VMEM((C, D), jnp.float32),
            pltpu.SemaphoreType.DMA,
            pltpu.SemaphoreType.DMA,
            pltpu.SemaphoreType.DMA,
            pltpu.SemaphoreType.DMA,
            pltpu.SemaphoreType.DMA,
            pltpu.SemaphoreType.DMA,
        ],
        compiler_params=pltpu.CompilerParams(use_tc_tiling_on_sc=False),
    )
    def k(ids_hbm, table_hbm, out_hbm, idx_v, rows0, rows1, rows2,
          sg0, sg1, sg2, ss0, ss1, ss2):
        wid = lax.axis_index("s") * _NC + lax.axis_index("c")
        base = wid * b_per_w
        pltpu.sync_copy(ids_hbm.at[pl.ds(base, b_per_w)], idx_v)

        rows, sg, ss = [rows0, rows1, rows2], [sg0, sg1, sg2], [ss0, ss1, ss2]
        gathers = [None] * 3
        stores = [None] * 3

        def gather_chunk(g):
            b = g % 3
            gathers[b] = pltpu.async_copy(
                table_hbm.at[idx_v.at[pl.ds(g * C, C)]], rows[b], sg[b])

        def store_chunk(g):
            b = g % 3
            stores[b] = pltpu.async_copy(
                rows[b], out_hbm.at[pl.ds(base + g * C, C)], ss[b])

        gather_chunk(0)
        gather_chunk(1)
        for g in range(n_chunks):
            b = g % 3
            if g + 2 < n_chunks:
                bn = (g + 2) % 3
                if stores[bn] is not None:
                    stores[bn].wait()
                gather_chunk(g + 2)
            gathers[b].wait()
            store_chunk(g)
        for b in range(3):
            if stores[b] is not None:
                stores[b].wait()

    return k


def kernel(input_ids, weight):
    bt, h = input_ids.shape
    v, d = weight.shape
    b = bt * h
    ids = input_ids.reshape(b).astype(jnp.int32)
    wt = jnp.swapaxes(weight, 0, 1)
    rem = v % 128
    if rem:
        aux = jnp.reshape(weight[v - rem:, :], (rem * d // 128, 128))
    else:
        aux = jnp.zeros((8, 128), jnp.float32)
    w2 = _make_transpose(v, d)(wt, aux)
    table = jnp.reshape(w2, (w2.shape[0] * 2, d))
    out = _make_gather(b, d, 512)(ids, table)
    return out.reshape(bt, h, d)
